# baseline (speedup 1.0000x reference)
import functools

import jax
import jax.numpy as jnp
from jax import lax
from jax.experimental import pallas as pl
from jax.experimental.pallas import tpu as pltpu
from jax.experimental.pallas import tpu_sc as plsc

_TOKENS = 32768
_D = 4096
_E = 64
_K = 8
_BT = 1024

_NC = 2
_NS = 16
_NW = _NC * _NS
_TPW = _TOKENS // _NW
_TB = 256


def _matmul_block(x_ref, w_ref, b_ref, out_ref):
    out_ref[...] = jax.lax.dot_general(
        x_ref[...], w_ref[...], (((1,), (1,)), ((), ())),
        preferred_element_type=jnp.float32,
    ) + b_ref[...]


def _tc_logits(x, W, b):
    b2 = b.reshape(1, _E)
    return pl.pallas_call(
        _matmul_block,
        grid=(_TOKENS // _BT,),
        in_specs=[
            pl.BlockSpec((_BT, _D), lambda i: (i, 0)),
            pl.BlockSpec((_E, _D), lambda i: (0, 0)),
            pl.BlockSpec((1, _E), lambda i: (0, 0)),
        ],
        out_specs=pl.BlockSpec((_BT, _E), lambda i: (i, 0)),
        out_shape=jax.ShapeDtypeStruct((_TOKENS, _E), jnp.float32),
        compiler_params=pltpu.CompilerParams(
            dimension_semantics=("arbitrary",),
        ),
    )(x, W, b2)


def _merge_desc(lane, ka, va, kb, vb):
    kb2 = lax.rev(kb, dimensions=(0,))
    vb2 = lax.rev(vb, dimensions=(0,))
    low = lane < 8
    kc = jnp.where(low, ka, kb2)
    vc = jnp.where(low, va, vb2)
    return plsc.sort_key_val(kc, vc, descending=True)


def _sc_body(logits_hbm, idx_hbm, probs_hbm, rows_v, idxb_v, probsb_v):
    wid = lax.axis_index("s") * _NC + lax.axis_index("c")
    base = wid * _TPW
    lane = lax.iota(jnp.int32, 16)
    low = lane < 8

    def block(bi, carry):
        b0 = base + bi * _TB
        pltpu.sync_copy(logits_hbm.at[pl.ds(b0, _TB)], rows_v)

        def tok(t, carry2):
            ks, vs = [], []
            for c in range(4):
                k = rows_v[t, pl.ds(c * 16, 16)]
                v = lane + (c * 16)
                k, v = plsc.sort_key_val(k, v, descending=True)
                ks.append(k)
                vs.append(v)
            k01, v01 = _merge_desc(lane, ks[0], vs[0], ks[1], vs[1])
            k23, v23 = _merge_desc(lane, ks[2], vs[2], ks[3], vs[3])
            kf, vf = _merge_desc(lane, k01, v01, k23, v23)
            m0 = jnp.max(kf)
            e = jnp.where(low, jnp.exp(kf - m0), jnp.float32(0.0))
            p = e / jnp.sum(e)
            tsplat = jnp.full((16,), t, jnp.int32)
            for c in range(4):
                probsb_v[t, pl.ds(c * 16, 16)] = jnp.zeros((16,), jnp.float32)
            plsc.store_scatter(probsb_v, [tsplat, vf], p, mask=low)
            plsc.store_scatter(idxb_v, [tsplat, lane], vf, mask=low)
            return carry2

        lax.fori_loop(0, _TB, tok, 0)
        pltpu.sync_copy(idxb_v, idx_hbm.at[pl.ds(b0, _TB)])
        pltpu.sync_copy(probsb_v, probs_hbm.at[pl.ds(b0, _TB)])
        return carry

    lax.fori_loop(0, _TPW // _TB, block, 0)


def _sc_topk(logits):
    mesh = plsc.VectorSubcoreMesh(
        core_axis_name="c", subcore_axis_name="s",
        num_cores=_NC, num_subcores=_NS,
    )
    return pl.kernel(
        _sc_body,
        out_type=[
            jax.ShapeDtypeStruct((_TOKENS, _K), jnp.int32),
            jax.ShapeDtypeStruct((_TOKENS, _E), jnp.float32),
        ],
        mesh=mesh,
        scratch_types=[
            pltpu.VMEM((_TB, _E), jnp.float32),
            pltpu.VMEM((_TB, _K), jnp.int32),
            pltpu.VMEM((_TB, _E), jnp.float32),
        ],
        compiler_params=pltpu.CompilerParams(needs_layout_passes=False),
    )(logits)


def kernel(x, W, b):
    logits = _tc_logits(x, W, b)
    idx, probs = _sc_topk(logits)
    return idx, probs
